# baseline (speedup 1.0000x reference)
import dataclasses
import functools
import math

import jax
import jax.numpy as jnp
from jax import lax
from jax.experimental import pallas as pl
from jax.experimental.pallas import tpu as pltpu
from jax.experimental.pallas import tpu_sc as plsc

_SIZE = 100000
_PAD = 0
_SMOOTHING = 0.1
_CONF = 1.0 - _SMOOTHING
_EPS = _SMOOTHING / (_SIZE - 2)
_C = (_SIZE - 2) * _EPS * math.log(_EPS) + _CONF * math.log(_CONF)

_N = 1024
_RB = 32
_NSTREAM = 4
_WAVE = 64


def _sc_gather(target, x):
    mesh = plsc.ScalarSubcoreMesh(axis_name="c", num_cores=2)
    half = _N // 2

    @functools.partial(
        pl.kernel,
        out_type=jax.ShapeDtypeStruct((_N,), jnp.float32),
        mesh=mesh,
        scratch_types=[
            pltpu.SMEM((half,), jnp.int32),
            pltpu.SMEM((_WAVE * 128,), jnp.float32),
            pltpu.SMEM((half,), jnp.float32),
            pltpu.SemaphoreType.DMA,
            pltpu.SemaphoreType.DMA,
        ],
    )
    def gather_kernel(t_hbm, x_hbm, out_hbm, idx_s, chunk_s, sel_s, sem,
                      gsem):
        cid = lax.axis_index("c")
        base = cid * half
        pltpu.async_copy(t_hbm.at[pl.ds(base, half)], idx_s, sem).wait()

        @pl.loop(0, half, step=_WAVE)
        def _(w):
            @pl.loop(0, _WAVE)
            def _(j):
                i = w + j
                t_al = pl.multiple_of((idx_s[i] >> 7) << 7, 128)
                pltpu.async_copy(
                    x_hbm.at[base + i].at[pl.ds(t_al, 128)],
                    chunk_s.at[pl.ds(j * 128, 128)], gsem)

            pltpu.make_async_copy(
                x_hbm.at[0].at[pl.ds(0, _WAVE * 128)], chunk_s, gsem).wait()

            @pl.loop(0, _WAVE)
            def _(j):
                i = w + j
                t = idx_s[i]
                sel_s[i] = chunk_s[j * 128 + (t & 127)]

        pltpu.async_copy(sel_s, out_hbm.at[pl.ds(base, half)], sem).wait()

    return gather_kernel(target, x)


def _rowsum_body(*refs):
    x_refs, (s_ref, x0_ref) = refs[:_NSTREAM], refs[_NSTREAM:]
    xs = [r[...] for r in x_refs]
    s_ref[...] = jnp.concatenate(
        [jnp.sum(xb, axis=1, keepdims=True) for xb in xs], axis=0)
    x0_ref[...] = jnp.concatenate([xb[:, 0:1] for xb in xs], axis=0)


def _tc_rowsum(x):
    stripe = _RB // _NSTREAM
    return pl.pallas_call(
        _rowsum_body,
        grid=(_N // _RB,),
        in_specs=[
            pl.BlockSpec((stripe, _SIZE),
                         lambda i, k=k: (i * _NSTREAM + k, 0))
            for k in range(_NSTREAM)
        ],
        out_specs=[
            pl.BlockSpec((_RB, 1), lambda i: (i, 0)),
            pl.BlockSpec((_RB, 1), lambda i: (i, 0)),
        ],
        out_shape=[
            jax.ShapeDtypeStruct((_N, 1), jnp.float32),
            jax.ShapeDtypeStruct((_N, 1), jnp.float32),
        ],
        compiler_params=pltpu.CompilerParams(
            dimension_semantics=("arbitrary",)),
    )(*([x] * _NSTREAM))


def _combine_body(s_ref, x0_ref, g_ref, t_ref, out_ref):
    s = s_ref[...]
    g = g_ref[...]
    x0 = x0_ref[...]
    t = t_ref[...]
    row = _C - _EPS * s + _EPS * x0 + (_EPS - _CONF) * g
    row = jnp.where(t != _PAD, row, 0.0)
    out_ref[...] = jnp.sum(row, keepdims=True)


def _tc_combine(s, x0, g2, t2):
    out = pl.pallas_call(
        _combine_body,
        out_shape=jax.ShapeDtypeStruct((1, 1), jnp.float32),
    )(s, x0, g2, t2)
    return out[0, 0]


def kernel(x, target):
    n, size = x.shape
    g = _sc_gather(target, x)
    s, x0 = _tc_rowsum(x)
    return _tc_combine(s, x0, g.reshape(n, 1), target.reshape(n, 1))


# speedup vs baseline: 1.0018x; 1.0018x over previous
import dataclasses
import functools
import math

import jax
import jax.numpy as jnp
from jax import lax
from jax.experimental import pallas as pl
from jax.experimental.pallas import tpu as pltpu
from jax.experimental.pallas import tpu_sc as plsc

_SIZE = 100000
_PAD = 0
_SMOOTHING = 0.1
_CONF = 1.0 - _SMOOTHING
_EPS = _SMOOTHING / (_SIZE - 2)
_C = (_SIZE - 2) * _EPS * math.log(_EPS) + _CONF * math.log(_CONF)

_N = 1024
_RB = 32
_NSTREAM = 4
_WAVE = 64


def _sc_gather(target, x):
    mesh = plsc.ScalarSubcoreMesh(axis_name="c", num_cores=2)
    half = _N // 2

    @functools.partial(
        pl.kernel,
        out_type=jax.ShapeDtypeStruct((_N,), jnp.float32),
        mesh=mesh,
        scratch_types=[
            pltpu.SMEM((half,), jnp.int32),
            pltpu.SMEM((_WAVE * 128,), jnp.float32),
            pltpu.SMEM((half,), jnp.float32),
            pltpu.SemaphoreType.DMA,
            pltpu.SemaphoreType.DMA,
        ],
    )
    def gather_kernel(t_hbm, x_hbm, out_hbm, idx_s, chunk_s, sel_s, sem,
                      gsem):
        cid = lax.axis_index("c")
        base = cid * half
        pltpu.async_copy(t_hbm.at[pl.ds(base, half)], idx_s, sem).wait()

        @pl.loop(0, half, step=_WAVE)
        def _(w):
            @pl.loop(0, _WAVE)
            def _(j):
                i = w + j
                t_al = pl.multiple_of((idx_s[i] >> 7) << 7, 128)
                pltpu.async_copy(
                    x_hbm.at[base + i].at[pl.ds(t_al, 128)],
                    chunk_s.at[pl.ds(j * 128, 128)], gsem)

            pltpu.make_async_copy(
                x_hbm.at[0].at[pl.ds(0, _WAVE * 128)], chunk_s, gsem).wait()

            @pl.loop(0, _WAVE)
            def _(j):
                i = w + j
                t = idx_s[i]
                sel_s[i] = chunk_s[j * 128 + (t & 127)]

        pltpu.async_copy(sel_s, out_hbm.at[pl.ds(base, half)], sem).wait()

    return gather_kernel(target, x)


def _rowsum_body(*refs):
    x_refs, (s_ref, x0_ref) = refs[:_NSTREAM], refs[_NSTREAM:]
    xs = [r[...] for r in x_refs]
    s_ref[...] = jnp.concatenate(
        [jnp.sum(xb, axis=1, keepdims=True) for xb in xs], axis=0)
    x0_ref[...] = jnp.concatenate([xb[:, 0:1] for xb in xs], axis=0)


def _tc_rowsum(x):
    stripe = _RB // _NSTREAM
    return pl.pallas_call(
        _rowsum_body,
        grid=(_N // _RB,),
        in_specs=[
            pl.BlockSpec((stripe, _SIZE),
                         lambda i, k=k: (i * _NSTREAM + k, 0))
            for k in range(_NSTREAM)
        ],
        out_specs=[
            pl.BlockSpec((_RB, 1), lambda i: (i, 0)),
            pl.BlockSpec((_RB, 1), lambda i: (i, 0)),
        ],
        out_shape=[
            jax.ShapeDtypeStruct((_N, 1), jnp.float32),
            jax.ShapeDtypeStruct((_N, 1), jnp.float32),
        ],
        compiler_params=pltpu.CompilerParams(
            dimension_semantics=("parallel",)),
    )(*([x] * _NSTREAM))


def _combine_body(s_ref, x0_ref, g_ref, t_ref, out_ref):
    s = s_ref[...]
    g = g_ref[...]
    x0 = x0_ref[...]
    t = t_ref[...]
    row = _C - _EPS * s + _EPS * x0 + (_EPS - _CONF) * g
    row = jnp.where(t != _PAD, row, 0.0)
    out_ref[...] = jnp.sum(row, keepdims=True)


def _tc_combine(s, x0, g2, t2):
    out = pl.pallas_call(
        _combine_body,
        out_shape=jax.ShapeDtypeStruct((1, 1), jnp.float32),
    )(s, x0, g2, t2)
    return out[0, 0]


def kernel(x, target):
    n, size = x.shape
    g = _sc_gather(target, x)
    s, x0 = _tc_rowsum(x)
    return _tc_combine(s, x0, g.reshape(n, 1), target.reshape(n, 1))
